# Initial kernel scaffold; baseline (speedup 1.0000x reference)
import jax
import jax.numpy as jnp
from jax.experimental import pallas as pl


def kernel(xyz, new_xyz, features):
    raise NotImplementedError("write your pallas kernel here")


# speedup vs baseline: 17.6863x; 17.6863x over previous
import functools

import jax
import jax.numpy as jnp
from jax import lax
from jax.experimental import pallas as pl
from jax.experimental.pallas import tpu as pltpu
from jax.experimental.pallas import tpu_sc as plsc

RADIUS = 0.2
NSAMPLE = 64
L = 16

B = 4
N = 16384
NPOINT = 1024
C = 128
COUT = C + 3

NW = 32
W_PER_B = NW // B
CPW = NPOINT // W_PER_B

GRP = 16
CHUNK = GRP * L
NCHUNKS = N // CHUNK
IDXBUF = NSAMPLE + CHUNK + L


def _sc_body(xyzt, cent, feat, out, xyz_v, cen_v, idxbuf, idxq, rows_v,
             stage, sem):
    wid = lax.axis_index("s") * 2 + lax.axis_index("c")
    b = wid // W_PER_B
    cbase = (wid % W_PER_B) * CPW

    pltpu.sync_copy(xyzt.at[b], xyz_v)
    for coord in range(3):
        pltpu.sync_copy(
            cent.at[b, pl.ds(coord * NPOINT + cbase, CPW)],
            cen_v.at[pl.ds(coord * CPW, CPW)])

    iota = lax.iota(jnp.int32, L)
    r2 = jnp.float32(RADIUS * RADIUS)

    def per_center(c, _):
        cvec = jnp.full((L,), c, jnp.int32)
        cx = plsc.load_gather(cen_v, [cvec])
        cy = plsc.load_gather(cen_v, [cvec + CPW])
        cz = plsc.load_gather(cen_v, [cvec + 2 * CPW])

        def chunk_cond(st):
            cnt, ci = st
            return jnp.logical_and(cnt < NSAMPLE, ci < NCHUNKS)

        def chunk_body(st):
            cnt, ci = st
            base = ci * CHUNK
            for t in range(GRP):
                off = base + t * L
                vx = xyz_v[pl.ds(off, L)]
                vy = xyz_v[pl.ds(N + off, L)]
                vz = xyz_v[pl.ds(2 * N + off, L)]
                dx = cx - vx
                dy = cy - vy
                dz = cz - vz
                d2 = dx * dx
                d2 = d2 + dy * dy
                d2 = d2 + dz * dz
                m = d2 < r2
                rank = plsc.cumsum(m.astype(jnp.int32))
                pos = rank + (cnt - 1)
                plsc.store_scatter(idxbuf, [pos], off + iota, mask=m)
                cnt = cnt + jnp.sum(m.astype(jnp.int32))
            return cnt, ci + 1

        cnt, _ = lax.while_loop(chunk_cond, chunk_body,
                                (jnp.int32(0), jnp.int32(0)))

        cnt_v = jnp.full((L,), cnt, jnp.int32)
        first = plsc.load_gather(idxbuf, [jnp.zeros((L,), jnp.int32)])
        first = jnp.where(cnt_v > 0, first, 0)
        idx_regs = []
        gbase = b * N
        for k in range(NSAMPLE // L):
            lane = k * L + iota
            v = idxbuf[pl.ds(k * L, L)]
            v = jnp.where(lane < cnt_v, v, first)
            idx_regs.append(v)
            idxq[pl.ds(k * L, L)] = v + gbase

        gather = pltpu.async_copy(feat.at[idxq], rows_v, sem)

        for k in range(NSAMPLE // L):
            gx = plsc.load_gather(xyz_v, [idx_regs[k]])
            gy = plsc.load_gather(xyz_v, [idx_regs[k] + N])
            gz = plsc.load_gather(xyz_v, [idx_regs[k] + 2 * N])
            stage[0, pl.ds(k * L, L)] = gx - cx
            stage[1, pl.ds(k * L, L)] = gy - cy
            stage[2, pl.ds(k * L, L)] = gz - cz

        gather.wait()

        def tr(ch, _):
            chv = jnp.full((L,), ch, jnp.int32)
            for k in range(NSAMPLE // L):
                samp = k * L + iota
                col = plsc.load_gather(rows_v, [samp, chv])
                plsc.store_scatter(stage, [chv + 3, samp], col)
            return 0

        lax.fori_loop(0, C, tr, 0)

        pltpu.sync_copy(stage, out.at[b, :, cbase + c, :])
        return 0

    lax.fori_loop(0, CPW, per_center, 0)


@jax.jit
def _run(xyzt, cent, feat):
    mesh = plsc.VectorSubcoreMesh(core_axis_name="c", subcore_axis_name="s")
    return pl.kernel(
        _sc_body,
        out_type=jax.ShapeDtypeStruct((B, COUT, NPOINT, NSAMPLE),
                                      jnp.float32),
        mesh=mesh,
        compiler_params=pltpu.CompilerParams(needs_layout_passes=False),
        scratch_types=[
            pltpu.VMEM((3 * N,), jnp.float32),
            pltpu.VMEM((3 * CPW,), jnp.float32),
            pltpu.VMEM((IDXBUF,), jnp.int32),
            pltpu.VMEM((NSAMPLE,), jnp.int32),
            pltpu.VMEM((NSAMPLE, C), jnp.float32),
            pltpu.VMEM((COUT, NSAMPLE), jnp.float32),
            pltpu.SemaphoreType.DMA,
        ],
    )(xyzt, cent, feat)


def kernel(xyz, new_xyz, features):
    xyzt = jnp.transpose(xyz, (0, 2, 1)).reshape(B, 3 * N)
    cent = jnp.transpose(new_xyz, (0, 2, 1)).reshape(B, 3 * NPOINT)
    feat = jnp.transpose(features, (0, 2, 1)).reshape(B * N, C)
    return _run(xyzt, cent, feat)
